# baseline (speedup 1.0000x reference)
import functools
import math

import jax
import jax.numpy as jnp
from jax.experimental import pallas as pl
from jax.experimental.pallas import tpu as pltpu

N = 4096
H = 256
W = 256

ROWS_PER_BAND = 8
PB = ROWS_PER_BAND * W
NB = 128
NBANDS = H // ROWS_PER_BAND
SQ2T = 5.2915
CLASS_SMAX = (2.0, 4.0, 6.0, 8.0)
NPAD = N + 8 * NB
CX = W * 0.5 + 0.5
CY = H * 0.5 + 0.5


def _params_kernel(p_ref, k_ref, fw_ref):
    x = p_ref[0:1, :]
    y = p_ref[1:2, :]
    sx = jnp.abs(p_ref[2:3, :])
    sy = jnp.abs(p_ref[3:4, :])
    rot = p_ref[4:5, :]
    f0 = p_ref[5:6, :]
    f1 = p_ref[6:7, :]
    f2 = p_ref[7:8, :]
    w = p_ref[8:9, :]

    mx = 0.5 * (x + 1.0) * W
    my = 0.5 * (y + 1.0) * H
    theta = jax.nn.sigmoid(rot) * (2.0 * math.pi)
    c = jnp.cos(theta)
    sn = jnp.sin(theta)
    sx2 = sx * sx
    sy2 = sy * sy
    Sxx = c * c * sx2 + sn * sn * sy2
    Sxy = c * sn * (sx2 - sy2)
    Syy = sn * sn * sx2 + c * c * sy2
    det = Sxx * Syy - Sxy * Sxy
    inv = 1.0 / (det + 1e-12)
    a = 0.5 * Syy * inv
    cc = -Sxy * inv
    b = 0.5 * Sxx * inv

    dmx = mx - CX
    dmy = my - CY
    k3 = -(2.0 * a * dmx + cc * dmy)
    k4 = -(2.0 * b * dmy + cc * dmx)
    k5 = a * dmx * dmx + b * dmy * dmy + cc * dmx * dmy

    zero = jnp.zeros_like(x)
    rows = [a, b, cc, k3, k4, k5, zero, zero]
    for i, r in enumerate(rows):
        k1 = r.astype(jnp.bfloat16)
        r1 = r - k1.astype(jnp.float32)
        k2 = r1.astype(jnp.bfloat16)
        r2 = r1 - k2.astype(jnp.float32)
        k3b = r2.astype(jnp.bfloat16)
        k_ref[i:i + 1, :] = k1
        k_ref[8 + i:9 + i, :] = k2
        k_ref[16 + i:17 + i, :] = k3b
        k_ref[24 + i:25 + i, :] = k1
        k_ref[32 + i:33 + i, :] = k2

    fw_ref[0:1, :] = f0 * w
    fw_ref[1:2, :] = f1 * w
    fw_ref[2:3, :] = f2 * w
    fw_ref[3:8, :] = jnp.concatenate([zero] * 5, axis=0)


def _raster_kernel(s_ref, k_ref, fw_ref, out_ref):
    i = pl.program_id(0)

    pix = jax.lax.broadcasted_iota(jnp.int32, (PB, 40), 0)
    lane = jax.lax.broadcasted_iota(jnp.int32, (PB, 40), 1)
    col = pix & (W - 1)
    row = pix >> 8
    pxi = col - (W // 2)
    pyi = row + i * ROWS_PER_BAND - (H // 2)
    qxx = pxi * pxi
    qyy = pyi * pyi
    qxy = pxi * pyi
    hxx = qxx & ~63
    hyy = qyy & ~63
    hxy = (qxy >> 6) << 6
    lxx = qxx - hxx
    lyy = qyy - hyy
    lxy = qxy - hxy
    m = lane & 7
    is_lo = lane >= 24
    fhi = jnp.where(m == 0, hxx,
          jnp.where(m == 1, hyy,
          jnp.where(m == 2, hxy,
          jnp.where(m == 3, pxi,
          jnp.where(m == 4, pyi,
          jnp.where(m == 5, 1, 0))))))
    flo = jnp.where(m == 0, lxx,
          jnp.where(m == 1, lyy,
          jnp.where(m == 2, lxy, 0)))
    Pf = jnp.where(is_lo, flo, fhi).astype(jnp.float32).astype(jnp.bfloat16)

    acc = jnp.zeros((PB, 8), jnp.float32)
    for c in range(4):
        lo = s_ref[i, 2 * c]
        hi = s_ref[i, 2 * c + 1]
        nblk = (hi - lo + NB - 1) // NB

        def body(j, acc, lo=lo):
            base = pl.multiple_of(lo + j * NB, NB)
            K = k_ref[:, pl.ds(base, NB)]
            sigma = jnp.dot(Pf, K, preferred_element_type=jnp.float32)
            vals = jnp.exp(-sigma).astype(jnp.bfloat16)
            fwb = fw_ref[pl.ds(base, NB), :]
            return acc + jnp.dot(vals, fwb, preferred_element_type=jnp.float32)

        acc = jax.lax.fori_loop(0, nblk, body, acc)

    out_ref[...] = jnp.clip(acc, 0.0, 1.0)


@jax.jit
def kernel(xyz, scaling, rotation, features, opacity):
    myf = 0.5 * (xyz[:, 1] + 1.0) * H
    s_max = jnp.maximum(jnp.abs(scaling[:, 0]), jnp.abs(scaling[:, 1]))
    cls = ((s_max > CLASS_SMAX[0]).astype(jnp.int32)
           + (s_max > CLASS_SMAX[1]).astype(jnp.int32)
           + (s_max > CLASS_SMAX[2]).astype(jnp.int32))
    key = cls.astype(jnp.float32) * 1024.0 + myf
    order = jnp.argsort(key)
    key_s = key[order]

    y0 = jnp.arange(NBANDS, dtype=jnp.float32) * ROWS_PER_BAND + 0.5
    y1 = y0 + (ROWS_PER_BAND - 1)
    Rc = jnp.array([SQ2T * s for s in CLASS_SMAX], jnp.float32)
    ckey = jnp.arange(4, dtype=jnp.float32) * 1024.0
    lo_q = ckey[None, :] + jnp.maximum(y0[:, None] - Rc[None, :], 0.0) - 1e-3
    hi_q = ckey[None, :] + jnp.minimum(y1[:, None] + Rc[None, :], 256.0) + 1e-3
    lo = jnp.searchsorted(key_s, lo_q.ravel()).astype(jnp.int32)
    hi = jnp.searchsorted(key_s, hi_q.ravel()).astype(jnp.int32)
    lo = lo.reshape(NBANDS, 4)
    hi = hi.reshape(NBANDS, 4)

    cnt = jnp.zeros((4,), jnp.int32).at[cls].add(1)
    first = jnp.concatenate([jnp.zeros((1,), jnp.int32),
                             jnp.cumsum(cnt)[:-1].astype(jnp.int32)])
    cap = ((cnt + NB - 1) // NB) * NB + NB
    segA = jnp.concatenate([jnp.zeros((1,), jnp.int32),
                            jnp.cumsum(cap)[:-1].astype(jnp.int32)])
    delta = segA - first
    lo = ((lo + delta[None, :]) // NB) * NB
    hi = hi + delta[None, :]
    scal = jnp.stack([lo[:, 0], hi[:, 0], lo[:, 1], hi[:, 1],
                      lo[:, 2], hi[:, 2], lo[:, 3], hi[:, 3]], axis=1)

    params = jnp.concatenate(
        [xyz.T, scaling.T, rotation.T, features.T, opacity.T,
         jnp.zeros((7, N), jnp.float32)], axis=0)
    cls_s = cls[order]
    pos = segA[cls_s] + jnp.arange(N, dtype=jnp.int32) - first[cls_s]
    imap = jnp.full((NPAD,), N, jnp.int32).at[pos].set(order)
    params = jnp.concatenate(
        [params, jnp.zeros((16, 1), jnp.float32)], axis=1)[:, imap]

    kcoef, fwT = pl.pallas_call(
        _params_kernel,
        out_shape=[jax.ShapeDtypeStruct((40, NPAD), jnp.bfloat16),
                   jax.ShapeDtypeStruct((8, NPAD), jnp.float32)],
    )(params)
    fw = fwT.T.astype(jnp.bfloat16)

    out = pl.pallas_call(
        _raster_kernel,
        grid_spec=pltpu.PrefetchScalarGridSpec(
            num_scalar_prefetch=1,
            grid=(NBANDS,),
            in_specs=[
                pl.BlockSpec((40, NPAD), lambda i, s: (0, 0)),
                pl.BlockSpec((NPAD, 8), lambda i, s: (0, 0)),
            ],
            out_specs=pl.BlockSpec((PB, 8), lambda i, s: (i, 0)),
        ),
        out_shape=jax.ShapeDtypeStruct((H * W, 8), jnp.float32),
    )(scal, kcoef, fw)

    img = out[:, :3].reshape(1, H, W, 3).transpose(0, 3, 1, 2)
    return img


# speedup vs baseline: 1.2034x; 1.2034x over previous
import functools
import math

import jax
import jax.numpy as jnp
from jax.experimental import pallas as pl
from jax.experimental.pallas import tpu as pltpu

N = 4096
H = 256
W = 256

ROWS_PER_BAND = 8
PB = ROWS_PER_BAND * W
NB = 256
NBANDS = H // ROWS_PER_BAND
SQ2T = 5.2915
CLASS_SMAX = (2.0, 4.0, 6.0, 8.0)
NPAD = N + NB
CX = W * 0.5 + 0.5
CY = H * 0.5 + 0.5


def _params_kernel(p_ref, k_ref, fw_ref):
    x = p_ref[0:1, :]
    y = p_ref[1:2, :]
    sx = jnp.abs(p_ref[2:3, :])
    sy = jnp.abs(p_ref[3:4, :])
    rot = p_ref[4:5, :]
    f0 = p_ref[5:6, :]
    f1 = p_ref[6:7, :]
    f2 = p_ref[7:8, :]
    w = p_ref[8:9, :]

    mx = 0.5 * (x + 1.0) * W
    my = 0.5 * (y + 1.0) * H
    theta = jax.nn.sigmoid(rot) * (2.0 * math.pi)
    c = jnp.cos(theta)
    sn = jnp.sin(theta)
    sx2 = sx * sx
    sy2 = sy * sy
    Sxx = c * c * sx2 + sn * sn * sy2
    Sxy = c * sn * (sx2 - sy2)
    Syy = sn * sn * sx2 + c * c * sy2
    det = Sxx * Syy - Sxy * Sxy
    inv = 1.0 / (det + 1e-12)
    a = 0.5 * Syy * inv
    cc = -Sxy * inv
    b = 0.5 * Sxx * inv

    dmx = mx - CX
    dmy = my - CY
    k3 = -(2.0 * a * dmx + cc * dmy)
    k4 = -(2.0 * b * dmy + cc * dmx)
    k5 = a * dmx * dmx + b * dmy * dmy + cc * dmx * dmy

    zero = jnp.zeros_like(x)
    rows = [a, b, cc, k3, k4, k5, zero, zero]
    for i, r in enumerate(rows):
        k1 = r.astype(jnp.bfloat16)
        r1 = r - k1.astype(jnp.float32)
        k2 = r1.astype(jnp.bfloat16)
        r2 = r1 - k2.astype(jnp.float32)
        k3b = r2.astype(jnp.bfloat16)
        k_ref[i:i + 1, :] = k1
        k_ref[8 + i:9 + i, :] = k2
        k_ref[16 + i:17 + i, :] = k3b
        k_ref[24 + i:25 + i, :] = k1
        k_ref[32 + i:33 + i, :] = k2

    fw_ref[0:1, :] = f0 * w
    fw_ref[1:2, :] = f1 * w
    fw_ref[2:3, :] = f2 * w
    fw_ref[3:8, :] = jnp.concatenate([zero] * 5, axis=0)


def _raster_kernel(s_ref, k_ref, fw_ref, out_ref):
    i = pl.program_id(0)

    pix = jax.lax.broadcasted_iota(jnp.int32, (PB, 40), 0)
    lane = jax.lax.broadcasted_iota(jnp.int32, (PB, 40), 1)
    col = pix & (W - 1)
    row = pix >> 8
    pxi = col - (W // 2)
    pyi = row + i * ROWS_PER_BAND - (H // 2)
    qxx = pxi * pxi
    qyy = pyi * pyi
    qxy = pxi * pyi
    hxx = qxx & ~63
    hyy = qyy & ~63
    hxy = (qxy >> 6) << 6
    lxx = qxx - hxx
    lyy = qyy - hyy
    lxy = qxy - hxy
    m = lane & 7
    is_lo = lane >= 24
    fhi = jnp.where(m == 0, hxx,
          jnp.where(m == 1, hyy,
          jnp.where(m == 2, hxy,
          jnp.where(m == 3, pxi,
          jnp.where(m == 4, pyi,
          jnp.where(m == 5, 1, 0))))))
    flo = jnp.where(m == 0, lxx,
          jnp.where(m == 1, lyy,
          jnp.where(m == 2, lxy, 0)))
    Pf = jnp.where(is_lo, flo, fhi).astype(jnp.float32).astype(jnp.bfloat16)

    gl = jax.lax.broadcasted_iota(jnp.int32, (1, NB), 1)

    acc = jnp.zeros((PB, 8), jnp.float32)
    for c in range(4):
        lo = s_ref[i, 2 * c]
        hi = s_ref[i, 2 * c + 1]
        nblk = (hi - lo + NB - 1) // NB

        def body(j, acc, lo=lo, hi=hi):
            base = pl.multiple_of(lo + j * NB, NB)
            K = k_ref[:, pl.ds(base, NB)]
            sigma = jnp.dot(Pf, K, preferred_element_type=jnp.float32)
            mask = (gl + base) < hi
            vals = jnp.where(mask, jnp.exp(-sigma), 0.0).astype(jnp.bfloat16)
            fwb = fw_ref[pl.ds(base, NB), :]
            return acc + jnp.dot(vals, fwb, preferred_element_type=jnp.float32)

        acc = jax.lax.fori_loop(0, nblk, body, acc)

    out_ref[...] = jnp.clip(acc, 0.0, 1.0)


@jax.jit
def kernel(xyz, scaling, rotation, features, opacity):
    myf = 0.5 * (xyz[:, 1] + 1.0) * H
    s_max = jnp.maximum(jnp.abs(scaling[:, 0]), jnp.abs(scaling[:, 1]))
    cls = ((s_max > CLASS_SMAX[0]).astype(jnp.int32)
           + (s_max > CLASS_SMAX[1]).astype(jnp.int32)
           + (s_max > CLASS_SMAX[2]).astype(jnp.int32))
    key = cls.astype(jnp.float32) * 1024.0 + myf
    order = jnp.argsort(key)
    key_s = key[order]

    y0 = jnp.arange(NBANDS, dtype=jnp.float32) * ROWS_PER_BAND + 0.5
    y1 = y0 + (ROWS_PER_BAND - 1)
    Rc = jnp.array([SQ2T * s for s in CLASS_SMAX], jnp.float32)
    ckey = jnp.arange(4, dtype=jnp.float32) * 1024.0
    lo_q = ckey[None, :] + jnp.maximum(y0[:, None] - Rc[None, :], 0.0) - 1e-3
    hi_q = ckey[None, :] + jnp.minimum(y1[:, None] + Rc[None, :], 256.0) + 1e-3
    lo = jnp.searchsorted(key_s, lo_q.ravel()).astype(jnp.int32)
    hi = jnp.searchsorted(key_s, hi_q.ravel()).astype(jnp.int32)
    lo = (lo.reshape(NBANDS, 4) // NB) * NB
    hi = hi.reshape(NBANDS, 4)
    scal = jnp.stack([lo[:, 0], hi[:, 0], lo[:, 1], hi[:, 1],
                      lo[:, 2], hi[:, 2], lo[:, 3], hi[:, 3]], axis=1)

    params = jnp.concatenate(
        [xyz.T, scaling.T, rotation.T, features.T, opacity.T,
         jnp.zeros((7, N), jnp.float32)], axis=0)
    params = params[:, order]
    params = jnp.concatenate(
        [params, jnp.zeros((16, NPAD - N), jnp.float32)], axis=1)

    kcoef, fwT = pl.pallas_call(
        _params_kernel,
        out_shape=[jax.ShapeDtypeStruct((40, NPAD), jnp.bfloat16),
                   jax.ShapeDtypeStruct((8, NPAD), jnp.float32)],
    )(params)
    fw = fwT.T.astype(jnp.bfloat16)

    out = pl.pallas_call(
        _raster_kernel,
        grid_spec=pltpu.PrefetchScalarGridSpec(
            num_scalar_prefetch=1,
            grid=(NBANDS,),
            in_specs=[
                pl.BlockSpec((40, NPAD), lambda i, s: (0, 0)),
                pl.BlockSpec((NPAD, 8), lambda i, s: (0, 0)),
            ],
            out_specs=pl.BlockSpec((PB, 8), lambda i, s: (i, 0)),
        ),
        out_shape=jax.ShapeDtypeStruct((H * W, 8), jnp.float32),
    )(scal, kcoef, fw)

    img = out[:, :3].reshape(1, H, W, 3).transpose(0, 3, 1, 2)
    return img


# speedup vs baseline: 1.2045x; 1.0010x over previous
import functools
import math

import jax
import jax.numpy as jnp
from jax.experimental import pallas as pl
from jax.experimental.pallas import tpu as pltpu

N = 4096
H = 256
W = 256

ROWS_PER_BAND = 8
PB = ROWS_PER_BAND * W
NB = 256
NBANDS = H // ROWS_PER_BAND
SQ2T = 5.2915
CLASS_SMAX = (2.0, 4.0, 6.0, 8.0)
NPAD = N + NB
CX = W * 0.5 + 0.5
CY = H * 0.5 + 0.5


def _params_kernel(p_ref, k_ref, fw_ref):
    x = p_ref[0:1, :]
    y = p_ref[1:2, :]
    sx = jnp.abs(p_ref[2:3, :])
    sy = jnp.abs(p_ref[3:4, :])
    rot = p_ref[4:5, :]
    f0 = p_ref[5:6, :]
    f1 = p_ref[6:7, :]
    f2 = p_ref[7:8, :]
    w = p_ref[8:9, :]

    mx = 0.5 * (x + 1.0) * W
    my = 0.5 * (y + 1.0) * H
    theta = jax.nn.sigmoid(rot) * (2.0 * math.pi)
    c = jnp.cos(theta)
    sn = jnp.sin(theta)
    sx2 = sx * sx
    sy2 = sy * sy
    Sxx = c * c * sx2 + sn * sn * sy2
    Sxy = c * sn * (sx2 - sy2)
    Syy = sn * sn * sx2 + c * c * sy2
    det = Sxx * Syy - Sxy * Sxy
    inv = 1.0 / (det + 1e-12)
    a = 0.5 * Syy * inv
    cc = -Sxy * inv
    b = 0.5 * Sxx * inv

    dmx = mx - CX
    dmy = my - CY
    k3 = -(2.0 * a * dmx + cc * dmy)
    k4 = -(2.0 * b * dmy + cc * dmx)
    k5 = a * dmx * dmx + b * dmy * dmy + cc * dmx * dmy

    zero = jnp.zeros_like(x)
    rows = [a, b, cc, k3, k4, k5, zero, zero]
    for i, r in enumerate(rows):
        k1 = r.astype(jnp.bfloat16)
        r1 = r - k1.astype(jnp.float32)
        k2 = r1.astype(jnp.bfloat16)
        r2 = r1 - k2.astype(jnp.float32)
        k3b = r2.astype(jnp.bfloat16)
        k_ref[i:i + 1, :] = k1
        k_ref[8 + i:9 + i, :] = k2
        k_ref[16 + i:17 + i, :] = k3b
        k_ref[24 + i:25 + i, :] = k1
        k_ref[32 + i:33 + i, :] = k2

    fw_ref[0:1, :] = f0 * w
    fw_ref[1:2, :] = f1 * w
    fw_ref[2:3, :] = f2 * w
    fw_ref[3:8, :] = jnp.concatenate([zero] * 5, axis=0)


def _raster_kernel(s_ref, k_ref, fw_ref, out_ref):
    i = pl.program_id(0)

    pix = jax.lax.broadcasted_iota(jnp.int32, (PB, 40), 0)
    lane = jax.lax.broadcasted_iota(jnp.int32, (PB, 40), 1)
    col = pix & (W - 1)
    row = pix >> 8
    pxi = col - (W // 2)
    pyi = row + i * ROWS_PER_BAND - (H // 2)
    qxx = pxi * pxi
    qyy = pyi * pyi
    qxy = pxi * pyi
    hxx = qxx & ~63
    hyy = qyy & ~63
    hxy = (qxy >> 6) << 6
    lxx = qxx - hxx
    lyy = qyy - hyy
    lxy = qxy - hxy
    m = lane & 7
    is_lo = lane >= 24
    fhi = jnp.where(m == 0, hxx,
          jnp.where(m == 1, hyy,
          jnp.where(m == 2, hxy,
          jnp.where(m == 3, pxi,
          jnp.where(m == 4, pyi,
          jnp.where(m == 5, 1, 0))))))
    flo = jnp.where(m == 0, lxx,
          jnp.where(m == 1, lyy,
          jnp.where(m == 2, lxy, 0)))
    Pf = jnp.where(is_lo, flo, fhi).astype(jnp.float32).astype(jnp.bfloat16)

    glc = jax.lax.broadcasted_iota(jnp.int32, (NB, 8), 0)

    acc = jnp.zeros((PB, 8), jnp.float32)
    for c in range(4):
        lo = s_ref[i, 2 * c]
        hi = s_ref[i, 2 * c + 1]
        nblk = (hi - lo + NB - 1) // NB

        def body(j, acc, lo=lo, hi=hi):
            base = pl.multiple_of(lo + j * NB, NB)
            K = k_ref[:, pl.ds(base, NB)]
            sigma = jnp.dot(Pf, K, preferred_element_type=jnp.float32)
            vals = jnp.exp(-sigma).astype(jnp.bfloat16)
            fwb = jnp.where((glc + base) < hi, fw_ref[pl.ds(base, NB), :],
                            jnp.bfloat16(0))
            return acc + jnp.dot(vals, fwb, preferred_element_type=jnp.float32)

        acc = jax.lax.fori_loop(0, nblk, body, acc)

    out_ref[...] = jnp.clip(acc, 0.0, 1.0)


@jax.jit
def kernel(xyz, scaling, rotation, features, opacity):
    myf = 0.5 * (xyz[:, 1] + 1.0) * H
    s_max = jnp.maximum(jnp.abs(scaling[:, 0]), jnp.abs(scaling[:, 1]))
    cls = ((s_max > CLASS_SMAX[0]).astype(jnp.int32)
           + (s_max > CLASS_SMAX[1]).astype(jnp.int32)
           + (s_max > CLASS_SMAX[2]).astype(jnp.int32))
    key = cls.astype(jnp.float32) * 1024.0 + myf
    order = jnp.argsort(key)
    key_s = key[order]

    y0 = jnp.arange(NBANDS, dtype=jnp.float32) * ROWS_PER_BAND + 0.5
    y1 = y0 + (ROWS_PER_BAND - 1)
    Rc = jnp.array([SQ2T * s for s in CLASS_SMAX], jnp.float32)
    ckey = jnp.arange(4, dtype=jnp.float32) * 1024.0
    lo_q = ckey[None, :] + jnp.maximum(y0[:, None] - Rc[None, :], 0.0) - 1e-3
    hi_q = ckey[None, :] + jnp.minimum(y1[:, None] + Rc[None, :], 256.0) + 1e-3
    lo = jnp.searchsorted(key_s, lo_q.ravel()).astype(jnp.int32)
    hi = jnp.searchsorted(key_s, hi_q.ravel()).astype(jnp.int32)
    lo = (lo.reshape(NBANDS, 4) // NB) * NB
    hi = hi.reshape(NBANDS, 4)
    scal = jnp.stack([lo[:, 0], hi[:, 0], lo[:, 1], hi[:, 1],
                      lo[:, 2], hi[:, 2], lo[:, 3], hi[:, 3]], axis=1)

    params = jnp.concatenate(
        [xyz.T, scaling.T, rotation.T, features.T, opacity.T,
         jnp.zeros((7, N), jnp.float32)], axis=0)
    params = params[:, order]
    params = jnp.concatenate(
        [params, jnp.zeros((16, NPAD - N), jnp.float32)], axis=1)

    kcoef, fwT = pl.pallas_call(
        _params_kernel,
        out_shape=[jax.ShapeDtypeStruct((40, NPAD), jnp.bfloat16),
                   jax.ShapeDtypeStruct((8, NPAD), jnp.float32)],
    )(params)
    fw = fwT.T.astype(jnp.bfloat16)

    out = pl.pallas_call(
        _raster_kernel,
        grid_spec=pltpu.PrefetchScalarGridSpec(
            num_scalar_prefetch=1,
            grid=(NBANDS,),
            in_specs=[
                pl.BlockSpec((40, NPAD), lambda i, s: (0, 0)),
                pl.BlockSpec((NPAD, 8), lambda i, s: (0, 0)),
            ],
            out_specs=pl.BlockSpec((PB, 8), lambda i, s: (i, 0)),
        ),
        out_shape=jax.ShapeDtypeStruct((H * W, 8), jnp.float32),
    )(scal, kcoef, fw)

    img = out[:, :3].reshape(1, H, W, 3).transpose(0, 3, 1, 2)
    return img
